# baseline (speedup 1.0000x reference)
import functools

import jax
import jax.numpy as jnp
from jax import lax
from jax.experimental import pallas as pl
from jax.experimental.pallas import tpu as pltpu
from jax.experimental.pallas import tpu_sc as plsc

N = 10000
E = 320000
D = 128
KS = [8000, 6400, 5120, 4096]

NC = 2
NS = 16
NW = NC * NS
CH = 128
ZR = 64
BR = 1024
NEG_INF = float("-inf")
IMIN = -2147483648

E_PAD = 327680


def _np_pad(n_rows):
    return ((n_rows + 2047) // 2048) * 2048


NP0 = _np_pad(N + 1)


def _agg_body(np_pad, n_ch, per_w, with_comp, *refs):
    if with_comp:
        (xp_hbm, src_hbm, dst_hbm, comp_hbm, out_hbm,
         sidxA, didxA, csrcA, cdstA, rowsA,
         sidxB, didxB, csrcB, cdstB, rowsB,
         zbuf_v, acc_sh, semA1, semA2, semA3, semB1, semB2, semB3) = refs
    else:
        (xp_hbm, src_hbm, dst_hbm, out_hbm,
         sidxA, didxA, csrcA, cdstA, rowsA,
         sidxB, didxB, csrcB, cdstB, rowsB,
         zbuf_v, acc_sh, semA1, semA2, semA3, semB1, semB2, semB3) = refs
    cid = lax.axis_index("c")
    sid = lax.axis_index("s")
    wid = sid * NC + cid

    def _zrow(i, _):
        def _zcol(j, __):
            zbuf_v[i, pl.ds(j * 16, 16)] = jnp.zeros((16,), jnp.float32)
            return __
        return lax.fori_loop(0, D // 16, _zcol, 0)
    lax.fori_loop(0, ZR, _zrow, 0)

    rt = np_pad // NS
    def _zfill(i, _):
        pltpu.sync_copy(zbuf_v, acc_sh.at[pl.ds(sid * rt + i * ZR, ZR)])
        return _
    lax.fori_loop(0, rt // ZR, _zfill, 0)
    plsc.subcore_barrier()

    base = wid * per_w

    def _edge2(t, _):
        offA = base + (2 * t) * CH
        offB = offA + CH
        pltpu.sync_copy(src_hbm.at[pl.ds(offA, CH)], sidxA)
        pltpu.sync_copy(dst_hbm.at[pl.ds(offA, CH)], didxA)
        pltpu.sync_copy(src_hbm.at[pl.ds(offB, CH)], sidxB)
        pltpu.sync_copy(dst_hbm.at[pl.ds(offB, CH)], didxB)
        if with_comp:
            cpAs = pltpu.async_copy(comp_hbm.at[sidxA], csrcA, semA1)
            cpAd = pltpu.async_copy(comp_hbm.at[didxA], cdstA, semA2)
            cpBs = pltpu.async_copy(comp_hbm.at[sidxB], csrcB, semB1)
            cpBd = pltpu.async_copy(comp_hbm.at[didxB], cdstB, semB2)
            cpAs.wait()
            rA = pltpu.async_copy(xp_hbm.at[csrcA], rowsA, semA3)
            cpBs.wait()
            rB = pltpu.async_copy(xp_hbm.at[csrcB], rowsB, semB3)
            cpAd.wait()
            rA.wait()
            pltpu.sync_copy(rowsA, acc_sh.at[cdstA], add=True)
            cpBd.wait()
            rB.wait()
            pltpu.sync_copy(rowsB, acc_sh.at[cdstB], add=True)
        else:
            rA = pltpu.async_copy(xp_hbm.at[sidxA], rowsA, semA3)
            rB = pltpu.async_copy(xp_hbm.at[sidxB], rowsB, semB3)
            rA.wait()
            pltpu.sync_copy(rowsA, acc_sh.at[didxA], add=True)
            rB.wait()
            pltpu.sync_copy(rowsB, acc_sh.at[didxB], add=True)
        return _
    lax.fori_loop(0, n_ch // 2, _edge2, 0)
    plsc.subcore_barrier()

    pltpu.sync_copy(acc_sh.at[pl.ds(sid * rt, rt)],
                    out_hbm.at[cid, pl.ds(sid * rt, rt)])


@functools.cache
def _make_agg(np_rows, with_comp):
    np_pad = _np_pad(np_rows)
    per_w = E_PAD // NW
    n_ch = per_w // CH
    mesh = plsc.VectorSubcoreMesh(core_axis_name="c", subcore_axis_name="s")
    body = functools.partial(_agg_body, np_pad, n_ch, per_w, with_comp)
    return pl.kernel(
        body,
        out_type=jax.ShapeDtypeStruct((NC, np_pad, D), jnp.float32),
        mesh=mesh,
        scratch_types=[
            pltpu.VMEM((CH,), jnp.int32),
            pltpu.VMEM((CH,), jnp.int32),
            pltpu.VMEM((CH,), jnp.int32),
            pltpu.VMEM((CH,), jnp.int32),
            pltpu.VMEM((CH, D), jnp.float32),
            pltpu.VMEM((CH,), jnp.int32),
            pltpu.VMEM((CH,), jnp.int32),
            pltpu.VMEM((CH,), jnp.int32),
            pltpu.VMEM((CH,), jnp.int32),
            pltpu.VMEM((CH, D), jnp.float32),
            pltpu.VMEM((ZR, D), jnp.float32),
            pltpu.VMEM_SHARED((np_pad, D), jnp.float32),
            pltpu.SemaphoreType.DMA,
            pltpu.SemaphoreType.DMA,
            pltpu.SemaphoreType.DMA,
            pltpu.SemaphoreType.DMA,
            pltpu.SemaphoreType.DMA,
            pltpu.SemaphoreType.DMA,
        ],
        name=f"gin_agg_{np_rows}_{int(with_comp)}",
    )


def _mlp_body(n, k_prev, G, xc_ref, agg_ref, W1_ref, b1_ref, W2_ref, b2_ref,
              x2_ref, *rest):
    if k_prev is not None:
        ro_ref = rest[0]
        ro_acc = rest[1]
    r = pl.program_id(0)
    xc = xc_ref[...]
    h = xc + (agg_ref[0] + agg_ref[1])
    y = jnp.maximum(jnp.dot(h, W1_ref[...],
                            preferred_element_type=jnp.float32) + b1_ref[...],
                    0.0)
    x2 = jnp.maximum(jnp.dot(y, W2_ref[...],
                             preferred_element_type=jnp.float32) + b2_ref[...],
                     0.0)
    x2_ref[...] = x2

    if k_prev is not None:
        rid = lax.broadcasted_iota(jnp.int32, (BR, 1), 0) + r * BR
        xm = jnp.where(rid < k_prev, xc, NEG_INF)
        bmax = jnp.max(xm, axis=0, keepdims=True)
        bsum = jnp.sum(xc, axis=0, keepdims=True)

        @pl.when(r == 0)
        def _init():
            ro_acc[0, :] = bmax[0]
            ro_acc[1, :] = bsum[0]

        @pl.when(r > 0)
        def _acc():
            ro_acc[0, :] = jnp.maximum(ro_acc[0, :], bmax[0])
            ro_acc[1, :] = ro_acc[1, :] + bsum[0]

        @pl.when(r == G - 1)
        def _fin():
            ro_ref[0, pl.ds(0, 128)] = ro_acc[0, :]
            ro_ref[0, pl.ds(128, 128)] = ro_acc[1, :] / float(k_prev)


@functools.cache
def _make_mlp(n, k_prev):
    np_pad = _np_pad(n + 1)
    G = np_pad // BR
    out_shapes = [jax.ShapeDtypeStruct((np_pad, D), jnp.float32)]
    out_specs = [pl.BlockSpec((BR, D), lambda r: (r, 0))]
    scratch = []
    if k_prev is not None:
        out_shapes.append(jax.ShapeDtypeStruct((1, 2 * D), jnp.float32))
        out_specs.append(pl.BlockSpec((1, 2 * D), lambda r: (0, 0)))
        scratch.append(pltpu.VMEM((2, D), jnp.float32))
    return pl.pallas_call(
        functools.partial(_mlp_body, n, k_prev, G),
        grid=(G,),
        in_specs=[
            pl.BlockSpec((BR, D), lambda r: (r, 0)),
            pl.BlockSpec((NC, BR, D), lambda r: (0, r, 0)),
            pl.BlockSpec((D, D), lambda r: (0, 0)),
            pl.BlockSpec((1, D), lambda r: (0, 0)),
            pl.BlockSpec((D, D), lambda r: (0, 0)),
            pl.BlockSpec((1, D), lambda r: (0, 0)),
        ],
        out_specs=out_specs,
        out_shape=out_shapes,
        scratch_shapes=scratch,
        name=f"gin_mlp_{n}",
    )


def _head_body(k4, G, xc_ref, ros_ref, l1W_ref, l1b_ref, l2W_ref, l2b_ref,
               l3W_ref, l3b_ref, out_ref, ro_acc):
    r = pl.program_id(0)
    xc = xc_ref[...]
    bmax = jnp.max(xc, axis=0, keepdims=True)
    bsum = jnp.sum(xc, axis=0, keepdims=True)

    @pl.when(r == 0)
    def _init():
        ro_acc[0, :] = bmax[0]
        ro_acc[1, :] = bsum[0]

    @pl.when(r > 0)
    def _acc():
        ro_acc[0, :] = jnp.maximum(ro_acc[0, :], bmax[0])
        ro_acc[1, :] = ro_acc[1, :] + bsum[0]

    @pl.when(r == G - 1)
    def _fin():
        ro4 = jnp.concatenate(
            [ro_acc[0, :].reshape(1, D),
             (ro_acc[1, :] / float(k4)).reshape(1, D)], axis=1)
        ros = ros_ref[...]
        h = ros[0:1] + ros[1:2] + ros[2:3] + ro4
        h = jnp.maximum(jnp.dot(h, l1W_ref[...],
                                preferred_element_type=jnp.float32)
                        + l1b_ref[...], 0.0)
        h = jnp.maximum(jnp.dot(h, l2W_ref[...],
                                preferred_element_type=jnp.float32)
                        + l2b_ref[...], 0.0)
        out_ref[...] = jnp.dot(h, l3W_ref[...],
                               preferred_element_type=jnp.float32) + l3b_ref[...]


@functools.cache
def _make_head(k4, np4, C):
    G = k4 // BR
    return pl.pallas_call(
        functools.partial(_head_body, k4, G),
        grid=(G,),
        in_specs=[
            pl.BlockSpec((BR, D), lambda r: (r, 0)),
            pl.BlockSpec((3, 2 * D), lambda r: (0, 0)),
            pl.BlockSpec((2 * D, D), lambda r: (0, 0)),
            pl.BlockSpec((1, D), lambda r: (0, 0)),
            pl.BlockSpec((D, D // 2), lambda r: (0, 0)),
            pl.BlockSpec((1, D // 2), lambda r: (0, 0)),
            pl.BlockSpec((D // 2, C), lambda r: (0, 0)),
            pl.BlockSpec((1, C), lambda r: (0, 0)),
        ],
        out_specs=pl.BlockSpec((1, C), lambda r: (0, 0)),
        out_shape=jax.ShapeDtypeStruct((1, C), jnp.float32),
        scratch_shapes=[pltpu.VMEM((2, D), jnp.float32)],
        name="gin_head",
    )


def kernel(x, edge_index, edge_attr, batch,
           W1a, b1a, W1b, b1b, p1,
           W2a, b2a, W2b, b2b, p2,
           W3a, b3a, W3b, b3b, p3,
           W4a, b4a, W4b, b4b, p4,
           l1W, l1b, l2W, l2b, l3W, l3b):
    del edge_attr, batch
    Ws = [(W1a, b1a, W1b, b1b, p1), (W2a, b2a, W2b, b2b, p2),
          (W3a, b3a, W3b, b3b, p3), (W4a, b4a, W4b, b4b, p4)]

    epad = jnp.full((E_PAD - E,), N, jnp.int32)
    src = jnp.concatenate([edge_index[0], epad])
    dst = jnp.concatenate([edge_index[1], epad])

    np_cur = _np_pad(N + 1)
    xc = jnp.zeros((np_cur, D), jnp.float32).at[:N].set(x)
    comp = None
    n = N
    ros = []
    for i in range(4):
        W1, b1, W2, b2, p = Ws[i]
        k = KS[i]
        if i == 0:
            agg = _make_agg(n + 1, False)(xc, src, dst)
        else:
            agg = _make_agg(n + 1, True)(xc, src, dst, comp)
        k_prev = None if i == 0 else KS[i - 1]
        mlp_outs = _make_mlp(n, k_prev)(
            xc, agg, W1, b1.reshape(1, D), W2, b2.reshape(1, D))
        x2 = mlp_outs[0]
        if k_prev is not None:
            ros.append(mlp_outs[1])
        score = jnp.tanh((x2[:n] @ p) / jnp.linalg.norm(p))
        vals, perm = lax.top_k(score, k)
        xnew = x2[perm] * vals[:, None]
        mapping = jnp.full((n + 1,), k, jnp.int32).at[perm].set(
            jnp.arange(k, dtype=jnp.int32))
        if i == 0:
            comp = jnp.concatenate(
                [jnp.arange(N, dtype=jnp.int32),
                 jnp.full((NP0 - N,), N, jnp.int32)])
        comp = mapping[comp]
        np_cur = _np_pad(k + 1)
        xc = jnp.zeros((np_cur, D), jnp.float32).at[:k].set(xnew)
        n = k

    ros_cat = jnp.concatenate(ros, axis=0)
    out = _make_head(KS[3], np_cur, 10)(
        xc, ros_cat, l1W, l1b.reshape(1, D), l2W, l2b.reshape(1, D // 2),
        l3W, l3b.reshape(1, 10))
    return out


# speedup vs baseline: 1.0505x; 1.0505x over previous
import functools

import jax
import jax.numpy as jnp
from jax import lax
from jax.experimental import pallas as pl
from jax.experimental.pallas import tpu as pltpu
from jax.experimental.pallas import tpu_sc as plsc

N = 10000
E = 320000
D = 128
KS = [8000, 6400, 5120, 4096]

NC = 2
NS = 16
NW = NC * NS
CH = 128
ZR = 64
BR = 1024
NEG_INF = float("-inf")
IMIN = -2147483648

E_PAD = 323584


def _np_pad(n_rows):
    return ((n_rows + 2047) // 2048) * 2048


NP0 = _np_pad(N + 1)


def _agg_body(np_pad, n_ch, per_w, with_comp, *refs):
    if with_comp:
        (xp_hbm, src_hbm, dst_hbm, comp_hbm, out_hbm,
         sidx_v, didx_v, csrc_v, cdst_v, rows_v, zbuf_v, acc_sh, sem) = refs
    else:
        (xp_hbm, src_hbm, dst_hbm, out_hbm,
         sidx_v, didx_v, csrc_v, cdst_v, rows_v, zbuf_v, acc_sh, sem) = refs
    cid = lax.axis_index("c")
    sid = lax.axis_index("s")
    wid = sid * NC + cid

    def _zrow(i, _):
        def _zcol(j, __):
            zbuf_v[i, pl.ds(j * 16, 16)] = jnp.zeros((16,), jnp.float32)
            return __
        return lax.fori_loop(0, D // 16, _zcol, 0)
    lax.fori_loop(0, ZR, _zrow, 0)

    rt = np_pad // NS
    def _zfill(i, _):
        pltpu.sync_copy(zbuf_v, acc_sh.at[pl.ds(sid * rt + i * ZR, ZR)])
        return _
    lax.fori_loop(0, rt // ZR, _zfill, 0)
    plsc.subcore_barrier()

    base = wid * per_w

    def _edge(c, _):
        off = base + c * CH
        pltpu.sync_copy(src_hbm.at[pl.ds(off, CH)], sidx_v)
        pltpu.sync_copy(dst_hbm.at[pl.ds(off, CH)], didx_v)
        if with_comp:
            pltpu.async_copy(comp_hbm.at[sidx_v], csrc_v, sem).wait()
            pltpu.async_copy(comp_hbm.at[didx_v], cdst_v, sem).wait()
            gsrc, gdst = csrc_v, cdst_v
        else:
            gsrc, gdst = sidx_v, didx_v
        pltpu.async_copy(xp_hbm.at[gsrc], rows_v, sem).wait()
        pltpu.sync_copy(rows_v, acc_sh.at[gdst], add=True)
        return _
    lax.fori_loop(0, n_ch, _edge, 0)
    plsc.subcore_barrier()

    pltpu.sync_copy(acc_sh.at[pl.ds(sid * rt, rt)],
                    out_hbm.at[cid, pl.ds(sid * rt, rt)])


@functools.cache
def _make_agg(np_rows, with_comp):
    np_pad = _np_pad(np_rows)
    per_w = E_PAD // NW
    n_ch = per_w // CH
    mesh = plsc.VectorSubcoreMesh(core_axis_name="c", subcore_axis_name="s")
    body = functools.partial(_agg_body, np_pad, n_ch, per_w, with_comp)
    return pl.kernel(
        body,
        out_type=jax.ShapeDtypeStruct((NC, np_pad, D), jnp.float32),
        mesh=mesh,
        scratch_types=[
            pltpu.VMEM((CH,), jnp.int32),
            pltpu.VMEM((CH,), jnp.int32),
            pltpu.VMEM((CH,), jnp.int32),
            pltpu.VMEM((CH,), jnp.int32),
            pltpu.VMEM((CH, D), jnp.float32),
            pltpu.VMEM((ZR, D), jnp.float32),
            pltpu.VMEM_SHARED((np_pad, D), jnp.float32),
            pltpu.SemaphoreType.DMA,
        ],
        name=f"gin_agg_{np_rows}_{int(with_comp)}",
    )


def _mlp_body(n, k_prev, G, xc_ref, agg_ref, W1_ref, b1_ref, W2_ref, b2_ref,
              x2_ref, *rest):
    if k_prev is not None:
        ro_ref = rest[0]
        ro_acc = rest[1]
    r = pl.program_id(0)
    xc = xc_ref[...]
    h = xc + (agg_ref[0] + agg_ref[1])
    y = jnp.maximum(jnp.dot(h, W1_ref[...],
                            preferred_element_type=jnp.float32) + b1_ref[...],
                    0.0)
    x2 = jnp.maximum(jnp.dot(y, W2_ref[...],
                             preferred_element_type=jnp.float32) + b2_ref[...],
                     0.0)
    x2_ref[...] = x2

    if k_prev is not None:
        rid = lax.broadcasted_iota(jnp.int32, (BR, 1), 0) + r * BR
        xm = jnp.where(rid < k_prev, xc, NEG_INF)
        bmax = jnp.max(xm, axis=0, keepdims=True)
        bsum = jnp.sum(xc, axis=0, keepdims=True)

        @pl.when(r == 0)
        def _init():
            ro_acc[0, :] = bmax[0]
            ro_acc[1, :] = bsum[0]

        @pl.when(r > 0)
        def _acc():
            ro_acc[0, :] = jnp.maximum(ro_acc[0, :], bmax[0])
            ro_acc[1, :] = ro_acc[1, :] + bsum[0]

        @pl.when(r == G - 1)
        def _fin():
            ro_ref[0, pl.ds(0, 128)] = ro_acc[0, :]
            ro_ref[0, pl.ds(128, 128)] = ro_acc[1, :] / float(k_prev)


@functools.cache
def _make_mlp(n, k_prev):
    np_pad = _np_pad(n + 1)
    G = np_pad // BR
    out_shapes = [jax.ShapeDtypeStruct((np_pad, D), jnp.float32)]
    out_specs = [pl.BlockSpec((BR, D), lambda r: (r, 0))]
    scratch = []
    if k_prev is not None:
        out_shapes.append(jax.ShapeDtypeStruct((1, 2 * D), jnp.float32))
        out_specs.append(pl.BlockSpec((1, 2 * D), lambda r: (0, 0)))
        scratch.append(pltpu.VMEM((2, D), jnp.float32))
    return pl.pallas_call(
        functools.partial(_mlp_body, n, k_prev, G),
        grid=(G,),
        in_specs=[
            pl.BlockSpec((BR, D), lambda r: (r, 0)),
            pl.BlockSpec((NC, BR, D), lambda r: (0, r, 0)),
            pl.BlockSpec((D, D), lambda r: (0, 0)),
            pl.BlockSpec((1, D), lambda r: (0, 0)),
            pl.BlockSpec((D, D), lambda r: (0, 0)),
            pl.BlockSpec((1, D), lambda r: (0, 0)),
        ],
        out_specs=out_specs,
        out_shape=out_shapes,
        scratch_shapes=scratch,
        name=f"gin_mlp_{n}",
    )


def _head_body(k4, G, xc_ref, ros_ref, l1W_ref, l1b_ref, l2W_ref, l2b_ref,
               l3W_ref, l3b_ref, out_ref, ro_acc):
    r = pl.program_id(0)
    xc = xc_ref[...]
    bmax = jnp.max(xc, axis=0, keepdims=True)
    bsum = jnp.sum(xc, axis=0, keepdims=True)

    @pl.when(r == 0)
    def _init():
        ro_acc[0, :] = bmax[0]
        ro_acc[1, :] = bsum[0]

    @pl.when(r > 0)
    def _acc():
        ro_acc[0, :] = jnp.maximum(ro_acc[0, :], bmax[0])
        ro_acc[1, :] = ro_acc[1, :] + bsum[0]

    @pl.when(r == G - 1)
    def _fin():
        ro4 = jnp.concatenate(
            [ro_acc[0, :].reshape(1, D),
             (ro_acc[1, :] / float(k4)).reshape(1, D)], axis=1)
        ros = ros_ref[...]
        h = ros[0:1] + ros[1:2] + ros[2:3] + ro4
        h = jnp.maximum(jnp.dot(h, l1W_ref[...],
                                preferred_element_type=jnp.float32)
                        + l1b_ref[...], 0.0)
        h = jnp.maximum(jnp.dot(h, l2W_ref[...],
                                preferred_element_type=jnp.float32)
                        + l2b_ref[...], 0.0)
        out_ref[...] = jnp.dot(h, l3W_ref[...],
                               preferred_element_type=jnp.float32) + l3b_ref[...]


@functools.cache
def _make_head(k4, np4, C):
    G = k4 // BR
    return pl.pallas_call(
        functools.partial(_head_body, k4, G),
        grid=(G,),
        in_specs=[
            pl.BlockSpec((BR, D), lambda r: (r, 0)),
            pl.BlockSpec((3, 2 * D), lambda r: (0, 0)),
            pl.BlockSpec((2 * D, D), lambda r: (0, 0)),
            pl.BlockSpec((1, D), lambda r: (0, 0)),
            pl.BlockSpec((D, D // 2), lambda r: (0, 0)),
            pl.BlockSpec((1, D // 2), lambda r: (0, 0)),
            pl.BlockSpec((D // 2, C), lambda r: (0, 0)),
            pl.BlockSpec((1, C), lambda r: (0, 0)),
        ],
        out_specs=pl.BlockSpec((1, C), lambda r: (0, 0)),
        out_shape=jax.ShapeDtypeStruct((1, C), jnp.float32),
        scratch_shapes=[pltpu.VMEM((2, D), jnp.float32)],
        name="gin_head",
    )


def kernel(x, edge_index, edge_attr, batch,
           W1a, b1a, W1b, b1b, p1,
           W2a, b2a, W2b, b2b, p2,
           W3a, b3a, W3b, b3b, p3,
           W4a, b4a, W4b, b4b, p4,
           l1W, l1b, l2W, l2b, l3W, l3b):
    del edge_attr, batch
    Ws = [(W1a, b1a, W1b, b1b, p1), (W2a, b2a, W2b, b2b, p2),
          (W3a, b3a, W3b, b3b, p3), (W4a, b4a, W4b, b4b, p4)]

    epad = jnp.full((E_PAD - E,), N, jnp.int32)
    src = jnp.concatenate([edge_index[0], epad])
    dst = jnp.concatenate([edge_index[1], epad])

    np_cur = _np_pad(N + 1)
    xc = jnp.zeros((np_cur, D), jnp.float32).at[:N].set(x)
    comp = None
    n = N
    ros = []
    for i in range(4):
        W1, b1, W2, b2, p = Ws[i]
        k = KS[i]
        if i == 0:
            agg = _make_agg(n + 1, False)(xc, src, dst)
        else:
            agg = _make_agg(n + 1, True)(xc, src, dst, comp)
        k_prev = None if i == 0 else KS[i - 1]
        mlp_outs = _make_mlp(n, k_prev)(
            xc, agg, W1, b1.reshape(1, D), W2, b2.reshape(1, D))
        x2 = mlp_outs[0]
        if k_prev is not None:
            ros.append(mlp_outs[1])
        score = jnp.tanh((x2[:n] @ p) / jnp.linalg.norm(p))
        vals, perm = lax.top_k(score, k)
        xnew = x2[perm] * vals[:, None]
        mapping = jnp.full((n + 1,), k, jnp.int32).at[perm].set(
            jnp.arange(k, dtype=jnp.int32))
        if i == 0:
            comp = jnp.concatenate(
                [jnp.arange(N, dtype=jnp.int32),
                 jnp.full((NP0 - N,), N, jnp.int32)])
        comp = mapping[comp]
        np_cur = _np_pad(k + 1)
        xc = jnp.zeros((np_cur, D), jnp.float32).at[:k].set(xnew)
        n = k

    ros_cat = jnp.concatenate(ros, axis=0)
    out = _make_head(KS[3], np_cur, 10)(
        xc, ros_cat, l1W, l1b.reshape(1, D), l2W, l2b.reshape(1, D // 2),
        l3W, l3b.reshape(1, 10))
    return out
